# Initial kernel scaffold; baseline (speedup 1.0000x reference)
import jax
import jax.numpy as jnp
from jax.experimental import pallas as pl


def kernel(x, stacked_params, layer_weights, edge_index):
    raise NotImplementedError("write your pallas kernel here")


# speedup vs baseline: 4.0313x; 4.0313x over previous
import functools

import jax
import jax.numpy as jnp
from jax import lax
from jax.experimental import pallas as pl
from jax.experimental.pallas import tpu as pltpu
from jax.experimental.pallas import tpu_sc as plsc

N = 10000
E = 320000
D = 128
L = 3
P = 8

NC = 2
NS = 16
NW = NC * NS
CH = 128
K = -(-E // (NW * CH))
EPW = K * CH
EPAD = NW * EPW
NACC = 10112
RPT = NACC // NS


def _sc_spmm_body(g_hbm, src_hbm, dst_hbm, zero_hbm, out_hbm,
                  src_v, dst_v, rows_v, acc_s, sem):
    c = lax.axis_index("c")
    s = lax.axis_index("s")
    wid = s * NC + c

    pltpu.sync_copy(zero_hbm, acc_s.at[pl.ds(s * RPT, RPT)])
    pltpu.sync_copy(src_hbm.at[wid], src_v)
    pltpu.sync_copy(dst_hbm.at[wid], dst_v)
    plsc.subcore_barrier()

    def body(j, carry):
        pltpu.async_copy(g_hbm.at[src_v.at[j]], rows_v, sem).wait()
        pltpu.sync_copy(rows_v, acc_s.at[dst_v.at[j]], add=True)
        return carry

    lax.fori_loop(0, K, body, 0)
    plsc.subcore_barrier()

    pltpu.sync_copy(acc_s.at[pl.ds(s * RPT, RPT)],
                    out_hbm.at[c].at[pl.ds(s * RPT, RPT)])


_sc_spmm = pl.kernel(
    _sc_spmm_body,
    out_type=jax.ShapeDtypeStruct((NC, NACC, D), jnp.float32),
    mesh=plsc.VectorSubcoreMesh(core_axis_name="c", subcore_axis_name="s",
                                num_cores=NC),
    scratch_types=[
        pltpu.VMEM((K, CH), jnp.int32),
        pltpu.VMEM((K, CH), jnp.int32),
        pltpu.VMEM((CH, D), jnp.float32),
        pltpu.VMEM_SHARED((NACC, D), jnp.float32),
        pltpu.SemaphoreType.DMA,
    ],
)


def _softmax_body(lw_ref, o_ref):
    v = lw_ref[...]
    m = jnp.max(v, axis=1, keepdims=True)
    e = jnp.exp(v - m)
    o_ref[...] = e / jnp.sum(e, axis=1, keepdims=True)


_softmax = pl.pallas_call(
    _softmax_body,
    out_shape=jax.ShapeDtypeStruct((L, P), jnp.float32),
)


def _combine_ww(sp_ref, w_ref):
    ww = w_ref[0] * sp_ref[0]
    for p in range(1, P):
        ww = ww + w_ref[p] * sp_ref[p]
    return ww


def _proj_body(h_ref, sp_ref, w_ref, o_ref):
    ww = _combine_ww(sp_ref, w_ref)
    o_ref[...] = lax.dot_general(h_ref[...], ww, (((1,), (1,)), ((), ())),
                                 preferred_element_type=jnp.float32)


def _fuse_body(p0_ref, p1_ref, sp_ref, w_ref, o_ref):
    h = jnp.maximum(p0_ref[...] + p1_ref[...], 0.0)
    ww = _combine_ww(sp_ref, w_ref)
    o_ref[...] = lax.dot_general(h, ww, (((1,), (1,)), ((), ())),
                                 preferred_element_type=jnp.float32)


def _add_body(p0_ref, p1_ref, o_ref):
    o_ref[...] = p0_ref[...] + p1_ref[...]


_vmem = pl.BlockSpec(memory_space=pltpu.VMEM)
_smem = pl.BlockSpec(memory_space=pltpu.SMEM)

_proj = pl.pallas_call(
    _proj_body,
    out_shape=jax.ShapeDtypeStruct((NACC, D), jnp.float32),
    in_specs=[_vmem, _vmem, _smem],
)

_fuse = pl.pallas_call(
    _fuse_body,
    out_shape=jax.ShapeDtypeStruct((NACC, D), jnp.float32),
    in_specs=[_vmem, _vmem, _vmem, _smem],
)

_add = pl.pallas_call(
    _add_body,
    out_shape=jax.ShapeDtypeStruct((NACC, D), jnp.float32),
    in_specs=[_vmem, _vmem],
)


def kernel(x, stacked_params, layer_weights, edge_index):
    pad = EPAD - E
    src = jnp.concatenate([edge_index[0], jnp.zeros((pad,), jnp.int32)])
    dst = jnp.concatenate([edge_index[1], jnp.full((pad,), N, jnp.int32)])
    srcr = src.reshape(NW, K, CH)
    dstr = dst.reshape(NW, K, CH)
    zero_blk = jnp.zeros((RPT, D), jnp.float32)
    spt = jnp.transpose(stacked_params, (0, 3, 1, 2))
    h = jnp.concatenate([x, jnp.zeros((NACC - N, D), jnp.float32)])

    wsm = _softmax(layer_weights)

    g = _proj(h, spt[0], wsm[0])
    parts = _sc_spmm(g, srcr, dstr, zero_blk)
    for i in range(1, L):
        g = _fuse(parts[0], parts[1], spt[i], wsm[i])
        parts = _sc_spmm(g, srcr, dstr, zero_blk)
    out = _add(parts[0], parts[1])
    return out[:N]
